# baseline (speedup 1.0000x reference)
import functools
import math

import numpy as np
import jax
import jax.numpy as jnp
from jax import lax
from jax.experimental import pallas as pl
from jax.experimental.pallas import tpu as pltpu
from jax.experimental.pallas import tpu_sc as plsc

_D = 64
_DP = 128
_MAX_LEN = 512


def _pe_np():
    pe = np.zeros((_MAX_LEN, _D), dtype=np.float32)
    position = np.arange(0.0, _MAX_LEN, dtype=np.float32)[:, None]
    div_term = np.exp(
        np.arange(0.0, _D, 2, dtype=np.float32) * -(math.log(10000.0) / _D))
    pe[:, 0::2] = np.sin(position * div_term)
    pe[:, 1::2] = np.cos(position * div_term)
    return pe


def _roles_tc(W, b, seq):
    pe = jnp.asarray(_pe_np()[:seq])
    scale = math.sqrt(_D)

    def body(pe_ref, w_ref, b_ref, out_ref):
        prod = lax.dot_general(
            pe_ref[...], w_ref[...],
            dimension_numbers=(((1,), (1,)), ((), ())),
            preferred_element_type=jnp.float32)
        out_ref[...] = (prod + b_ref[...] + 1.0) * scale

    return pl.pallas_call(
        body,
        out_shape=jax.ShapeDtypeStruct((seq, _D), jnp.float32),
    )(pe, W, b.reshape(1, _D))


@functools.partial(jax.jit, static_argnums=(3, 4))
def _sc_gather_mul(xf, table128, roles_f, n_ch, ch):
    seq = roles_f.shape[0] // _D
    total = xf.shape[0]
    info = plsc.get_sparse_core_info()
    nc, ns = info.num_cores, info.num_subcores
    nw = nc * ns
    mesh = plsc.VectorSubcoreMesh(core_axis_name="c", subcore_axis_name="s")
    b_per_w = n_ch * ch
    assert b_per_w * nw == total

    nbuf = 4
    ahead = 3
    assert n_ch % nbuf == 0 and ahead < nbuf

    @functools.partial(
        pl.kernel,
        mesh=mesh,
        out_type=jax.ShapeDtypeStruct((total, _DP), jnp.float32),
        scratch_types=[
            pltpu.VMEM((b_per_w,), jnp.int32),
            pltpu.VMEM((2 * seq * _D,), jnp.float32),
            [pltpu.VMEM((ch, _DP), jnp.float32)] * nbuf,
            [pltpu.SemaphoreType.DMA] * nbuf,
            [pltpu.SemaphoreType.DMA] * nbuf,
        ],
    )
    def k(x_hbm, table_hbm, roles_hbm, out_hbm, idx_v, roles_v, rows, gsems,
          ssems):
        wid = lax.axis_index("s") * nc + lax.axis_index("c")
        base = wid * b_per_w
        pltpu.sync_copy(x_hbm.at[pl.ds(base, b_per_w)], idx_v)
        pltpu.sync_copy(roles_hbm, roles_v.at[pl.ds(0, seq * _D)])
        pltpu.sync_copy(roles_hbm, roles_v.at[pl.ds(seq * _D, seq * _D)])

        def gather(c, b):
            return pltpu.async_copy(
                table_hbm.at[idx_v.at[pl.ds(c * ch, ch)]], rows[b], gsems[b])

        def store(c, b):
            return pltpu.async_copy(rows[b], out_hbm.at[pl.ds(base + c * ch,
                                                              ch)], ssems[b])

        for cc in range(ahead):
            gather(cc, cc % nbuf)

        def group(g, carry):
            for b in range(nbuf):
                cc = g * nbuf + b
                pltpu.make_async_copy(
                    table_hbm.at[idx_v.at[pl.ds(cc * ch, ch)]], rows[b],
                    gsems[b]).wait()
                p0 = lax.rem(cc * ch, seq)

                @plsc.parallel_loop(0, ch)
                def _(i):
                    r0 = (p0 + i) * _D
                    for j in range(_D // 16):
                        sl = pl.ds(j * 16, 16)
                        rows[b][i, sl] = (rows[b][i, sl] *
                                          roles_v[pl.ds(r0 + j * 16, 16)])

                store(cc, b)
                cn = cc + ahead
                b2 = (b + ahead) % nbuf

                @pl.when(cn < n_ch)
                def _():
                    @pl.when(cc >= nbuf - ahead)
                    def _():
                        pltpu.make_async_copy(
                            rows[b2], out_hbm.at[pl.ds(base, ch)],
                            ssems[b2]).wait()

                    gather(cn, b2)
            return carry

        lax.fori_loop(0, n_ch // nbuf, group, 0)
        for b in range(nbuf):
            pltpu.make_async_copy(rows[b], out_hbm.at[pl.ds(base, ch)],
                                  ssems[b]).wait()

    return k(xf, table128, roles_f)


def kernel(x, table, W, b):
    batch, seq = x.shape
    d = table.shape[1]
    roles = _roles_tc(W, b, seq)
    total = batch * seq
    nw = 32
    ch = 128
    b_per_w = total // nw
    n_ch = b_per_w // ch
    xf = x.reshape(-1).astype(jnp.int32)
    table128 = jnp.pad(table, ((0, 0), (0, _DP - d)))
    out128 = _sc_gather_mul(xf, table128, roles.reshape(-1), n_ch, ch)
    return out128[:, :d].reshape(batch, seq, d)


# speedup vs baseline: 1.0042x; 1.0042x over previous
import functools
import math

import numpy as np
import jax
import jax.numpy as jnp
from jax import lax
from jax.experimental import pallas as pl
from jax.experimental.pallas import tpu as pltpu
from jax.experimental.pallas import tpu_sc as plsc

_D = 64
_DP = 128
_MAX_LEN = 512


def _pe_np():
    pe = np.zeros((_MAX_LEN, _D), dtype=np.float32)
    position = np.arange(0.0, _MAX_LEN, dtype=np.float32)[:, None]
    div_term = np.exp(
        np.arange(0.0, _D, 2, dtype=np.float32) * -(math.log(10000.0) / _D))
    pe[:, 0::2] = np.sin(position * div_term)
    pe[:, 1::2] = np.cos(position * div_term)
    return pe


def _roles_tc(W, b, seq):
    pe = jnp.asarray(_pe_np()[:seq])
    scale = math.sqrt(_D)

    def body(pe_ref, w_ref, b_ref, out_ref):
        prod = lax.dot_general(
            pe_ref[...], w_ref[...],
            dimension_numbers=(((1,), (1,)), ((), ())),
            preferred_element_type=jnp.float32)
        out_ref[...] = (prod + b_ref[...] + 1.0) * scale

    return pl.pallas_call(
        body,
        out_shape=jax.ShapeDtypeStruct((seq, _D), jnp.float32),
    )(pe, W, b.reshape(1, _D))


@functools.partial(jax.jit, static_argnums=(3, 4))
def _sc_gather_mul(xf, table128, roles_f, n_ch, ch):
    seq = roles_f.shape[0] // _D
    total = xf.shape[0]
    info = plsc.get_sparse_core_info()
    nc, ns = info.num_cores, info.num_subcores
    nw = nc * ns
    mesh = plsc.VectorSubcoreMesh(core_axis_name="c", subcore_axis_name="s")
    b_per_w = n_ch * ch
    assert b_per_w * nw == total

    nbuf = 4
    ahead = 3
    assert n_ch % nbuf == 0 and ahead < nbuf

    @functools.partial(
        pl.kernel,
        mesh=mesh,
        out_type=jax.ShapeDtypeStruct((total, _DP), jnp.float32),
        scratch_types=[
            pltpu.VMEM((b_per_w,), jnp.int32),
            pltpu.VMEM((2 * seq * _D,), jnp.float32),
            [pltpu.VMEM((ch, _DP), jnp.float32)] * nbuf,
            [pltpu.SemaphoreType.DMA] * nbuf,
            [pltpu.SemaphoreType.DMA] * nbuf,
        ],
    )
    def k(x_hbm, table_hbm, roles_hbm, out_hbm, idx_v, roles_v, rows, gsems,
          ssems):
        wid = lax.axis_index("s") * nc + lax.axis_index("c")
        base = wid * b_per_w
        pltpu.sync_copy(x_hbm.at[pl.ds(base, b_per_w)], idx_v)
        pltpu.sync_copy(roles_hbm, roles_v.at[pl.ds(0, seq * _D)])
        pltpu.sync_copy(roles_hbm, roles_v.at[pl.ds(seq * _D, seq * _D)])

        def gather(c, b):
            return pltpu.async_copy(
                table_hbm.at[idx_v.at[pl.ds(c * ch, ch)]], rows[b], gsems[b])

        def store(c, b):
            return pltpu.async_copy(rows[b], out_hbm.at[pl.ds(base + c * ch,
                                                              ch)], ssems[b])

        for cc in range(ahead):
            gather(cc, cc % nbuf)

        def group(g, carry):
            for b in range(nbuf):
                cc = g * nbuf + b
                pltpu.make_async_copy(
                    table_hbm.at[idx_v.at[pl.ds(cc * ch, ch)]], rows[b],
                    gsems[b]).wait()
                p0 = lax.rem(cc * ch, seq)

                @plsc.parallel_loop(0, ch, unroll=4)
                def _(i):
                    r0 = (p0 + i) * _D
                    for j in range(_D // 16):
                        sl = pl.ds(j * 16, 16)
                        rows[b][i, sl] = (rows[b][i, sl] *
                                          roles_v[pl.ds(r0 + j * 16, 16)])

                store(cc, b)
                cn = cc + ahead
                b2 = (b + ahead) % nbuf

                @pl.when(cn < n_ch)
                def _():
                    @pl.when(cc >= nbuf - ahead)
                    def _():
                        pltpu.make_async_copy(
                            rows[b2], out_hbm.at[pl.ds(base, ch)],
                            ssems[b2]).wait()

                    gather(cn, b2)
            return carry

        lax.fori_loop(0, n_ch // nbuf, group, 0)
        for b in range(nbuf):
            pltpu.make_async_copy(rows[b], out_hbm.at[pl.ds(base, ch)],
                                  ssems[b]).wait()

    return k(xf, table128, roles_f)


def kernel(x, table, W, b):
    batch, seq = x.shape
    d = table.shape[1]
    roles = _roles_tc(W, b, seq)
    total = batch * seq
    nw = 32
    ch = 128
    b_per_w = total // nw
    n_ch = b_per_w // ch
    xf = x.reshape(-1).astype(jnp.int32)
    table128 = jnp.pad(table, ((0, 0), (0, _DP - d)))
    out128 = _sc_gather_mul(xf, table128, roles.reshape(-1), n_ch, ch)
    return out128[:, :d].reshape(batch, seq, d)
